# baseline (speedup 1.0000x reference)
import dataclasses
import functools

import jax
import jax.numpy as jnp
from jax import lax
from jax.experimental import pallas as pl
from jax.experimental.pallas import tpu as pltpu
from jax.experimental.pallas import tpu_sc as plsc

N_NODES = 10000
N_EDGES = 320000
D = 128

NP = 10112
EPW = N_EDGES // 32
NW = 32
CH = 128
G = 8
NGRP = 10
NCHUNK = G * NGRP
PADW = NCHUNK * CH - EPW
E_PAD = NW * NCHUNK * CH
RPT = NP // 16
NBLK = NP // 128
HBLK = 40
HROWS = HBLK * 16

_SC_CP = pltpu.CompilerParams()
if "needs_layout_passes" in pltpu.CompilerParams.__dataclass_fields__:
    _SC_CP = dataclasses.replace(_SC_CP, needs_layout_passes=False)

_MESH = plsc.VectorSubcoreMesh(core_axis_name="c", subcore_axis_name="s")


def _sc_degree(didx, zhist):
    @functools.partial(
        pl.kernel,
        out_type=jax.ShapeDtypeStruct((NW, 2, HROWS * 128), jnp.float32),
        mesh=_MESH,
        compiler_params=_SC_CP,
        scratch_types=[
            pltpu.VMEM((NCHUNK, CH), jnp.int32),
            pltpu.VMEM((HROWS * 128,), jnp.float32),
            pltpu.SemaphoreType.DMA,
        ],
    )
    def k(didx_hbm, zhist_hbm, out_hbm, didx_v, hist_v, sem):
        c = lax.axis_index("c")
        s = lax.axis_index("s")
        wid = s * jnp.int32(2) + c
        pltpu.sync_copy(didx_hbm.at[wid], didx_v)
        lanes128 = lax.iota(jnp.int32, 16) * jnp.int32(128)
        ones16 = jnp.ones((16,), jnp.float32)
        for p in range(2):
            blo = jnp.int32(p * HBLK)
            span = jnp.uint32((HBLK if p == 0 else NBLK) - p * HBLK)
            pltpu.sync_copy(zhist_hbm, hist_v)

            def body(j, carry):
                for kk in range(CH // 16):
                    idx16 = didx_v[j, pl.ds(kk * 16, 16)]
                    rel = lax.shift_right_logical(idx16, jnp.int32(7)) - blo
                    m = plsc.bitcast(rel, jnp.uint32) < span
                    addr = (lax.shift_left(rel, jnp.int32(11)) + lanes128
                            + lax.bitwise_and(idx16, jnp.int32(127)))
                    plsc.addupdate_scatter(hist_v, [addr], ones16, mask=m)
                return carry

            lax.fori_loop(jnp.int32(0), jnp.int32(NCHUNK), body, 0)
            pltpu.sync_copy(hist_v, out_hbm.at[wid, jnp.int32(p)])

    return k(didx, zhist)


def _sc_propagate(sidx, didx, yn, zrows):
    @functools.partial(
        pl.kernel,
        out_type=jax.ShapeDtypeStruct((2, NP, D), jnp.float32),
        mesh=_MESH,
        scratch_types=[
            pltpu.VMEM((NCHUNK, CH), jnp.int32),
            pltpu.VMEM((G, CH), jnp.int32),
            pltpu.VMEM((CH, D), jnp.float32),
            pltpu.VMEM((CH, D), jnp.float32),
            pltpu.VMEM_SHARED((NP, D), jnp.float32),
            pltpu.SemaphoreType.DMA,
            pltpu.SemaphoreType.DMA,
            pltpu.SemaphoreType.DMA,
            pltpu.SemaphoreType.DMA,
        ],
    )
    def k(sidx_hbm, didx_hbm, yn_hbm, zrows_hbm, out_hbm,
          sidx_v, dg_v, rows_a, rows_b, acc_sh, semi, semd, semg0, semg1):
        c = lax.axis_index("c")
        s = lax.axis_index("s")
        wid = s * jnp.int32(2) + c
        cps = pltpu.async_copy(sidx_hbm.at[wid], sidx_v, semi)
        pltpu.async_copy(didx_hbm.at[wid, pl.ds(jnp.int32(0), G)], dg_v, semd)

        pltpu.sync_copy(zrows_hbm, acc_sh.at[pl.ds(s * jnp.int32(RPT), RPT)])
        cps.wait()
        plsc.subcore_barrier()
        pltpu.async_copy(yn_hbm.at[sidx_v.at[jnp.int32(0)]], rows_a, semg0)

        def body(g, carry):
            base = g * jnp.int32(G)
            pltpu.make_async_copy(didx_hbm.at[wid, pl.ds(jnp.int32(0), G)],
                                  dg_v, semd).wait()
            for k_ in range(G):
                j = base + jnp.int32(k_)
                jn = j + jnp.int32(1)
                cur, semc = (rows_a, semg0) if k_ % 2 == 0 else (rows_b, semg1)
                nxt, semn = (rows_b, semg1) if k_ % 2 == 0 else (rows_a, semg0)

                @pl.when(jn < jnp.int32(NCHUNK))
                def _():
                    pltpu.async_copy(yn_hbm.at[sidx_v.at[jn]], nxt, semn)

                pltpu.make_async_copy(yn_hbm.at[sidx_v.at[j]], cur, semc).wait()
                pltpu.sync_copy(cur, acc_sh.at[dg_v.at[jnp.int32(k_)]], add=True)

            @pl.when(g + jnp.int32(1) < jnp.int32(NGRP))
            def _():
                pltpu.async_copy(
                    didx_hbm.at[wid,
                                pl.ds((g + jnp.int32(1)) * jnp.int32(G), G)],
                    dg_v, semd)

            return carry

        lax.fori_loop(jnp.int32(0), jnp.int32(NGRP), body, 0)

        plsc.subcore_barrier()
        pltpu.sync_copy(acc_sh.at[pl.ds(s * jnp.int32(RPT), RPT)],
                        out_hbm.at[c, pl.ds(s * jnp.int32(RPT), RPT)])

    return k(sidx, didx, yn, zrows)


def _tc_prep(degh, yp, xp, alp, lam):
    def body(alp_ref, lam_ref, degh_ref, y_ref, x_ref, yn_ref, base_ref,
             snorm_ref):
        a = alp_ref[0, 0]
        l = lam_ref[0, 0]
        dh2 = degh_ref[...]
        d0 = jnp.sum(dh2[:, 0].reshape(NW, HBLK, 16, 128), axis=(0, 2))
        d1 = jnp.sum(dh2[:, 1].reshape(NW, HBLK, 16, 128), axis=(0, 2))
        deg = jnp.concatenate([d0, d1], axis=0)[:NBLK]
        dn = l * deg + (1.0 - l)
        norm = lax.rsqrt(dn)
        y3 = y_ref[...].reshape(NBLK, 128, D)
        yn_ref[...] = (y3 * norm[:, :, None]).reshape(NP, D)
        x3 = x_ref[...].reshape(NBLK, 128, D)
        base3 = (1.0 - a) * y3 + (a / dn)[:, :, None] * x3
        base_ref[...] = base3.reshape(NP, D)
        snorm_ref[...] = (a * l) * norm

    return pl.pallas_call(
        body,
        out_shape=[
            jax.ShapeDtypeStruct((NP, D), jnp.float32),
            jax.ShapeDtypeStruct((NP, D), jnp.float32),
            jax.ShapeDtypeStruct((NBLK, 128), jnp.float32),
        ],
    )(alp, lam, degh, yp, xp)


def _tc_final(zp, base, snorm):
    def body(zp_ref, base_ref, snorm_ref, out_ref):
        z3 = (zp_ref[0] + zp_ref[1]).reshape(NBLK, 128, D)
        out3 = z3 * snorm_ref[...][:, :, None]
        out_ref[...] = base_ref[...] + out3.reshape(NP, D)

    return pl.pallas_call(
        body,
        out_shape=jax.ShapeDtypeStruct((NP, D), jnp.float32),
    )(zp, base, snorm)


@jax.jit
def kernel(edge_index, Y, X, alp, lam):
    src = edge_index[0].astype(jnp.int32)
    dst = edge_index[1].astype(jnp.int32)
    fill = jnp.broadcast_to(
        N_NODES + jnp.arange(PADW, dtype=jnp.int32) % (NP - N_NODES),
        (NW, PADW))
    sidx = jnp.concatenate([src.reshape(NW, EPW), fill], axis=1)
    sidx = sidx.reshape(NW, NCHUNK, CH)
    didx = jnp.concatenate([dst.reshape(NW, EPW), fill], axis=1)
    didx = didx.reshape(NW, NCHUNK, CH)
    yp = jnp.pad(Y.astype(jnp.float32), ((0, NP - N_NODES), (0, 0)))
    xp = jnp.pad(X.astype(jnp.float32), ((0, NP - N_NODES), (0, 0)))
    alp2 = jnp.reshape(alp.astype(jnp.float32), (1, 1))
    lam2 = jnp.reshape(lam.astype(jnp.float32), (1, 1))

    zhist = jnp.zeros((HROWS * 128,), jnp.float32)
    zrows = jnp.zeros((RPT, D), jnp.float32)

    degh = _sc_degree(didx, zhist).reshape(NW, 2, HROWS, 128)
    yn, base, snorm = _tc_prep(degh, yp, xp, alp2, lam2)
    zp = _sc_propagate(sidx, didx, yn, zrows)
    out = _tc_final(zp, base, snorm)
    return out[:N_NODES]


# speedup vs baseline: 1.1313x; 1.1313x over previous
import dataclasses
import functools

import jax
import jax.numpy as jnp
from jax import lax
from jax.experimental import pallas as pl
from jax.experimental.pallas import tpu as pltpu
from jax.experimental.pallas import tpu_sc as plsc

N_NODES = 10000
N_EDGES = 320000
D = 128

NP = 10112
EPW = N_EDGES // 32
NW = 32
CH = 128
G = 8
NGRP = 10
NCHUNK = G * NGRP
PADW = NCHUNK * CH - EPW
E_PAD = NW * NCHUNK * CH
RPT = NP // 16
NBLK = NP // 128
HBLK = 40
HROWS = HBLK * 16

_SC_CP = pltpu.CompilerParams()
if "needs_layout_passes" in pltpu.CompilerParams.__dataclass_fields__:
    _SC_CP = dataclasses.replace(_SC_CP, needs_layout_passes=False)

_MESH = plsc.VectorSubcoreMesh(core_axis_name="c", subcore_axis_name="s")


def _sc_degree(didx, zhist):
    @functools.partial(
        pl.kernel,
        out_type=jax.ShapeDtypeStruct((NW, 2, HROWS, 128), jnp.float32),
        mesh=_MESH,
        compiler_params=_SC_CP,
        scratch_types=[
            pltpu.VMEM((NCHUNK, CH), jnp.int32),
            pltpu.VMEM((HROWS, 128), jnp.float32),
            pltpu.SemaphoreType.DMA,
        ],
    )
    def k(didx_hbm, zhist_hbm, out_hbm, didx_v, hist_v, sem):
        c = lax.axis_index("c")
        s = lax.axis_index("s")
        wid = s * jnp.int32(2) + c
        pltpu.sync_copy(didx_hbm.at[wid], didx_v)
        lanes = lax.iota(jnp.int32, 16)
        ones16 = jnp.ones((16,), jnp.float32)
        for p in range(2):
            blo = jnp.int32(p * HBLK)
            bhi = jnp.int32(HBLK if p == 0 else NBLK)
            pltpu.sync_copy(zhist_hbm, hist_v)

            def body(j, carry):
                for kk in range(CH // 16):
                    idx16 = didx_v[j, pl.ds(kk * 16, 16)]
                    blk = lax.shift_right_logical(idx16, jnp.int32(7))
                    m = (blk >= blo) & (blk < bhi)
                    row = (blk - blo) * jnp.int32(16) + lanes
                    col = lax.bitwise_and(idx16, jnp.int32(127))
                    plsc.addupdate_scatter(hist_v, [row, col], ones16, mask=m)
                return carry

            lax.fori_loop(jnp.int32(0), jnp.int32(NCHUNK), body, 0)
            pltpu.sync_copy(hist_v, out_hbm.at[wid, jnp.int32(p)])

    return k(didx, zhist)


def _sc_propagate(sidx, didx, yn, zrows):
    @functools.partial(
        pl.kernel,
        out_type=jax.ShapeDtypeStruct((2, NP, D), jnp.float32),
        mesh=_MESH,
        scratch_types=[
            pltpu.VMEM((NCHUNK, CH), jnp.int32),
            pltpu.VMEM((G, CH), jnp.int32),
            pltpu.VMEM((CH, D), jnp.float32),
            pltpu.VMEM((CH, D), jnp.float32),
            pltpu.VMEM_SHARED((NP, D), jnp.float32),
            pltpu.SemaphoreType.DMA,
            pltpu.SemaphoreType.DMA,
            pltpu.SemaphoreType.DMA,
            pltpu.SemaphoreType.DMA,
        ],
    )
    def k(sidx_hbm, didx_hbm, yn_hbm, zrows_hbm, out_hbm,
          sidx_v, dg_v, rows_a, rows_b, acc_sh, semi, semd, semg0, semg1):
        c = lax.axis_index("c")
        s = lax.axis_index("s")
        wid = s * jnp.int32(2) + c
        cps = pltpu.async_copy(sidx_hbm.at[wid], sidx_v, semi)
        pltpu.async_copy(didx_hbm.at[wid, pl.ds(jnp.int32(0), G)], dg_v, semd)

        pltpu.sync_copy(zrows_hbm, acc_sh.at[pl.ds(s * jnp.int32(RPT), RPT)])
        cps.wait()
        plsc.subcore_barrier()
        pltpu.async_copy(yn_hbm.at[sidx_v.at[jnp.int32(0)]], rows_a, semg0)

        def body(g, carry):
            base = g * jnp.int32(G)
            pltpu.make_async_copy(didx_hbm.at[wid, pl.ds(jnp.int32(0), G)],
                                  dg_v, semd).wait()
            for k_ in range(G):
                j = base + jnp.int32(k_)
                jn = j + jnp.int32(1)
                cur, semc = (rows_a, semg0) if k_ % 2 == 0 else (rows_b, semg1)
                nxt, semn = (rows_b, semg1) if k_ % 2 == 0 else (rows_a, semg0)

                @pl.when(jn < jnp.int32(NCHUNK))
                def _():
                    pltpu.async_copy(yn_hbm.at[sidx_v.at[jn]], nxt, semn)

                pltpu.make_async_copy(yn_hbm.at[sidx_v.at[j]], cur, semc).wait()
                pltpu.sync_copy(cur, acc_sh.at[dg_v.at[jnp.int32(k_)]], add=True)

            @pl.when(g + jnp.int32(1) < jnp.int32(NGRP))
            def _():
                pltpu.async_copy(
                    didx_hbm.at[wid,
                                pl.ds((g + jnp.int32(1)) * jnp.int32(G), G)],
                    dg_v, semd)

            return carry

        lax.fori_loop(jnp.int32(0), jnp.int32(NGRP), body, 0)

        plsc.subcore_barrier()
        pltpu.sync_copy(acc_sh.at[pl.ds(s * jnp.int32(RPT), RPT)],
                        out_hbm.at[c, pl.ds(s * jnp.int32(RPT), RPT)])

    return k(sidx, didx, yn, zrows)


def _tc_prep(degh, yp, xp, alp, lam):
    def body(alp_ref, lam_ref, degh_ref, y_ref, x_ref, yn_ref, base_ref,
             snorm_ref):
        a = alp_ref[0, 0]
        l = lam_ref[0, 0]
        dh2 = degh_ref[...]
        d0 = jnp.sum(dh2[:, 0].reshape(NW, HBLK, 16, 128), axis=(0, 2))
        d1 = jnp.sum(dh2[:, 1].reshape(NW, HBLK, 16, 128), axis=(0, 2))
        deg = jnp.concatenate([d0, d1], axis=0)[:NBLK]
        dn = l * deg + (1.0 - l)
        norm = lax.rsqrt(dn)
        y3 = y_ref[...].reshape(NBLK, 128, D)
        yn_ref[...] = (y3 * norm[:, :, None]).reshape(NP, D)
        x3 = x_ref[...].reshape(NBLK, 128, D)
        base3 = (1.0 - a) * y3 + (a / dn)[:, :, None] * x3
        base_ref[...] = base3.reshape(NP, D)
        snorm_ref[...] = (a * l) * norm

    return pl.pallas_call(
        body,
        out_shape=[
            jax.ShapeDtypeStruct((NP, D), jnp.float32),
            jax.ShapeDtypeStruct((NP, D), jnp.float32),
            jax.ShapeDtypeStruct((NBLK, 128), jnp.float32),
        ],
    )(alp, lam, degh, yp, xp)


def _tc_final(zp, base, snorm):
    def body(zp_ref, base_ref, snorm_ref, out_ref):
        z3 = (zp_ref[0] + zp_ref[1]).reshape(NBLK, 128, D)
        out3 = z3 * snorm_ref[...][:, :, None]
        out_ref[...] = base_ref[...] + out3.reshape(NP, D)

    return pl.pallas_call(
        body,
        out_shape=jax.ShapeDtypeStruct((NP, D), jnp.float32),
    )(zp, base, snorm)


@jax.jit
def kernel(edge_index, Y, X, alp, lam):
    src = edge_index[0].astype(jnp.int32)
    dst = edge_index[1].astype(jnp.int32)
    fill = jnp.broadcast_to(
        N_NODES + jnp.arange(PADW, dtype=jnp.int32) % (NP - N_NODES),
        (NW, PADW))
    sidx = jnp.concatenate([src.reshape(NW, EPW), fill], axis=1)
    sidx = sidx.reshape(NW, NCHUNK, CH)
    didx = jnp.concatenate([dst.reshape(NW, EPW), fill], axis=1)
    didx = didx.reshape(NW, NCHUNK, CH)
    yp = jnp.pad(Y.astype(jnp.float32), ((0, NP - N_NODES), (0, 0)))
    xp = jnp.pad(X.astype(jnp.float32), ((0, NP - N_NODES), (0, 0)))
    alp2 = jnp.reshape(alp.astype(jnp.float32), (1, 1))
    lam2 = jnp.reshape(lam.astype(jnp.float32), (1, 1))

    zhist = jnp.zeros((HROWS, 128), jnp.float32)
    zrows = jnp.zeros((RPT, D), jnp.float32)

    degh = _sc_degree(didx, zhist)
    yn, base, snorm = _tc_prep(degh, yp, xp, alp2, lam2)
    zp = _sc_propagate(sidx, didx, yn, zrows)
    out = _tc_final(zp, base, snorm)
    return out[:N_NODES]


# speedup vs baseline: 1.1540x; 1.0201x over previous
import dataclasses
import functools

import jax
import jax.numpy as jnp
from jax import lax
from jax.experimental import pallas as pl
from jax.experimental.pallas import tpu as pltpu
from jax.experimental.pallas import tpu_sc as plsc

N_NODES = 10000
N_EDGES = 320000
D = 128

NP = 10112
EPW = N_EDGES // 32
NW = 32
CH = 128
G = 4
NGRP = 20
NCHUNK = G * NGRP
PADW = NCHUNK * CH - EPW
E_PAD = NW * NCHUNK * CH
RPT = NP // 16
NBLK = NP // 128
HBLK = 40
HROWS = HBLK * 16

_SC_CP = pltpu.CompilerParams()
if "needs_layout_passes" in pltpu.CompilerParams.__dataclass_fields__:
    _SC_CP = dataclasses.replace(_SC_CP, needs_layout_passes=False)

_MESH = plsc.VectorSubcoreMesh(core_axis_name="c", subcore_axis_name="s")


def _sc_degree(didx, zhist):
    @functools.partial(
        pl.kernel,
        out_type=jax.ShapeDtypeStruct((NW, 2, HROWS, 128), jnp.float32),
        mesh=_MESH,
        compiler_params=_SC_CP,
        scratch_types=[
            pltpu.VMEM((NCHUNK, CH), jnp.int32),
            pltpu.VMEM((HROWS, 128), jnp.float32),
            pltpu.SemaphoreType.DMA,
        ],
    )
    def k(didx_hbm, zhist_hbm, out_hbm, didx_v, hist_v, sem):
        c = lax.axis_index("c")
        s = lax.axis_index("s")
        wid = s * jnp.int32(2) + c
        pltpu.sync_copy(didx_hbm.at[wid], didx_v)
        lanes = lax.iota(jnp.int32, 16)
        ones16 = jnp.ones((16,), jnp.float32)
        for p in range(2):
            blo = jnp.int32(p * HBLK)
            bhi = jnp.int32(HBLK if p == 0 else NBLK)
            pltpu.sync_copy(zhist_hbm, hist_v)

            def body(j, carry):
                for kk in range(CH // 16):
                    idx16 = didx_v[j, pl.ds(kk * 16, 16)]
                    blk = lax.shift_right_logical(idx16, jnp.int32(7))
                    m = (blk >= blo) & (blk < bhi)
                    row = (blk - blo) * jnp.int32(16) + lanes
                    col = lax.bitwise_and(idx16, jnp.int32(127))
                    plsc.addupdate_scatter(hist_v, [row, col], ones16, mask=m)
                return carry

            lax.fori_loop(jnp.int32(0), jnp.int32(NCHUNK), body, 0)
            pltpu.sync_copy(hist_v, out_hbm.at[wid, jnp.int32(p)])

    return k(didx, zhist)


def _sc_propagate(sidx, didx, yn, zrows):
    @functools.partial(
        pl.kernel,
        out_type=jax.ShapeDtypeStruct((2, NP, D), jnp.float32),
        mesh=_MESH,
        scratch_types=[
            pltpu.VMEM((NCHUNK, CH), jnp.int32),
            pltpu.VMEM((G, CH), jnp.int32),
            pltpu.VMEM((G, CH), jnp.int32),
            pltpu.VMEM((CH, D), jnp.float32),
            pltpu.VMEM((CH, D), jnp.float32),
            pltpu.VMEM_SHARED((NP, D), jnp.float32),
            pltpu.SemaphoreType.DMA,
            pltpu.SemaphoreType.DMA,
            pltpu.SemaphoreType.DMA,
            pltpu.SemaphoreType.DMA,
            pltpu.SemaphoreType.DMA,
        ],
    )
    def k(sidx_hbm, didx_hbm, yn_hbm, zrows_hbm, out_hbm,
          sidx_v, dg0_v, dg1_v, rows_a, rows_b, acc_sh,
          semi, semd0, semd1, semg0, semg1):
        c = lax.axis_index("c")
        s = lax.axis_index("s")
        wid = s * jnp.int32(2) + c
        cps = pltpu.async_copy(sidx_hbm.at[wid], sidx_v, semi)
        pltpu.async_copy(didx_hbm.at[wid, pl.ds(jnp.int32(0), G)], dg0_v, semd0)
        pltpu.async_copy(didx_hbm.at[wid, pl.ds(jnp.int32(G), G)], dg1_v, semd1)

        pltpu.sync_copy(zrows_hbm, acc_sh.at[pl.ds(s * jnp.int32(RPT), RPT)])
        cps.wait()
        plsc.subcore_barrier()
        pltpu.async_copy(yn_hbm.at[sidx_v.at[jnp.int32(0)]], rows_a, semg0)

        def body(u, carry):
            for gg in range(2):
                g = u * jnp.int32(2) + jnp.int32(gg)
                base = g * jnp.int32(G)
                dg_v, semd = (dg0_v, semd0) if gg == 0 else (dg1_v, semd1)
                pltpu.make_async_copy(
                    didx_hbm.at[wid, pl.ds(jnp.int32(0), G)], dg_v,
                    semd).wait()
                for k_ in range(G):
                    j = base + jnp.int32(k_)
                    jn = j + jnp.int32(1)
                    cur, semc = ((rows_a, semg0) if k_ % 2 == 0
                                 else (rows_b, semg1))
                    nxt, semn = ((rows_b, semg1) if k_ % 2 == 0
                                 else (rows_a, semg0))

                    @pl.when(jn < jnp.int32(NCHUNK))
                    def _():
                        pltpu.async_copy(yn_hbm.at[sidx_v.at[jn]], nxt, semn)

                    pltpu.make_async_copy(yn_hbm.at[sidx_v.at[j]], cur,
                                          semc).wait()
                    pltpu.sync_copy(cur, acc_sh.at[dg_v.at[jnp.int32(k_)]],
                                    add=True)

                @pl.when(g + jnp.int32(2) < jnp.int32(NGRP))
                def _():
                    pltpu.async_copy(
                        didx_hbm.at[wid,
                                    pl.ds((g + jnp.int32(2)) * jnp.int32(G),
                                          G)],
                        dg_v, semd)

            return carry

        lax.fori_loop(jnp.int32(0), jnp.int32(NGRP // 2), body, 0)

        plsc.subcore_barrier()
        pltpu.sync_copy(acc_sh.at[pl.ds(s * jnp.int32(RPT), RPT)],
                        out_hbm.at[c, pl.ds(s * jnp.int32(RPT), RPT)])

    return k(sidx, didx, yn, zrows)


def _tc_prep(degh, yp, xp, alp, lam):
    def body(alp_ref, lam_ref, degh_ref, y_ref, x_ref, yn_ref, base_ref,
             snorm_ref):
        a = alp_ref[0, 0]
        l = lam_ref[0, 0]
        dh2 = degh_ref[...]
        d0 = jnp.sum(dh2[:, 0].reshape(NW, HBLK, 16, 128), axis=(0, 2))
        d1 = jnp.sum(dh2[:, 1].reshape(NW, HBLK, 16, 128), axis=(0, 2))
        deg = jnp.concatenate([d0, d1], axis=0)[:NBLK]
        dn = l * deg + (1.0 - l)
        norm = lax.rsqrt(dn)
        y3 = y_ref[...].reshape(NBLK, 128, D)
        yn_ref[...] = (y3 * norm[:, :, None]).reshape(NP, D)
        x3 = x_ref[...].reshape(NBLK, 128, D)
        base3 = (1.0 - a) * y3 + (a / dn)[:, :, None] * x3
        base_ref[...] = base3.reshape(NP, D)
        snorm_ref[...] = (a * l) * norm

    return pl.pallas_call(
        body,
        out_shape=[
            jax.ShapeDtypeStruct((NP, D), jnp.float32),
            jax.ShapeDtypeStruct((NP, D), jnp.float32),
            jax.ShapeDtypeStruct((NBLK, 128), jnp.float32),
        ],
    )(alp, lam, degh, yp, xp)


def _tc_final(zp, base, snorm):
    def body(zp_ref, base_ref, snorm_ref, out_ref):
        z3 = (zp_ref[0] + zp_ref[1]).reshape(NBLK, 128, D)
        out3 = z3 * snorm_ref[...][:, :, None]
        out_ref[...] = base_ref[...] + out3.reshape(NP, D)

    return pl.pallas_call(
        body,
        out_shape=jax.ShapeDtypeStruct((NP, D), jnp.float32),
    )(zp, base, snorm)


@jax.jit
def kernel(edge_index, Y, X, alp, lam):
    src = edge_index[0].astype(jnp.int32)
    dst = edge_index[1].astype(jnp.int32)
    fill = jnp.broadcast_to(
        N_NODES + jnp.arange(PADW, dtype=jnp.int32) % (NP - N_NODES),
        (NW, PADW))
    sidx = jnp.concatenate([src.reshape(NW, EPW), fill], axis=1)
    sidx = sidx.reshape(NW, NCHUNK, CH)
    didx = jnp.concatenate([dst.reshape(NW, EPW), fill], axis=1)
    didx = didx.reshape(NW, NCHUNK, CH)
    yp = jnp.pad(Y.astype(jnp.float32), ((0, NP - N_NODES), (0, 0)))
    xp = jnp.pad(X.astype(jnp.float32), ((0, NP - N_NODES), (0, 0)))
    alp2 = jnp.reshape(alp.astype(jnp.float32), (1, 1))
    lam2 = jnp.reshape(lam.astype(jnp.float32), (1, 1))

    zhist = jnp.zeros((HROWS, 128), jnp.float32)
    zrows = jnp.zeros((RPT, D), jnp.float32)

    degh = _sc_degree(didx, zhist)
    yn, base, snorm = _tc_prep(degh, yp, xp, alp2, lam2)
    zp = _sc_propagate(sidx, didx, yn, zrows)
    out = _tc_final(zp, base, snorm)
    return out[:N_NODES]
